# Initial kernel scaffold; baseline (speedup 1.0000x reference)
import jax
import jax.numpy as jnp
from jax.experimental import pallas as pl


def kernel(x, weight):
    raise NotImplementedError("write your pallas kernel here")


# speedup vs baseline: 1.5492x; 1.5492x over previous
import functools

import jax
import jax.numpy as jnp
from jax import lax
from jax.experimental import pallas as pl
from jax.experimental.pallas import tpu as pltpu
from jax.experimental.pallas import tpu_sc as plsc

EMBED_DIM = 32
B_TOTAL = 16384 * 26
NUM_CORES = 2
NUM_SUBCORES = 16
NUM_WORKERS = NUM_CORES * NUM_SUBCORES
ROWS_PER_WORKER = B_TOTAL // NUM_WORKERS
CHUNK = 1024
NCHUNKS = ROWS_PER_WORKER // CHUNK


def _build():
    mesh = plsc.VectorSubcoreMesh(core_axis_name="c", subcore_axis_name="s")

    @functools.partial(
        pl.kernel,
        mesh=mesh,
        out_type=jax.ShapeDtypeStruct((B_TOTAL, EMBED_DIM), jnp.float32),
        scratch_types=[
            pltpu.VMEM((CHUNK,), jnp.int32),
            pltpu.VMEM((CHUNK, EMBED_DIM), jnp.float32),
            pltpu.SemaphoreType.DMA,
        ],
        compiler_params=pltpu.CompilerParams(use_tc_tiling_on_sc=False),
    )
    def body(x_ref, w_ref, out_ref, idx_v, rows_v, sem):
        wid = lax.axis_index("s") * NUM_CORES + lax.axis_index("c")
        base0 = wid * ROWS_PER_WORKER

        def chunk_body(i, carry):
            base = base0 + i * CHUNK
            pltpu.sync_copy(x_ref.at[pl.ds(base, CHUNK)], idx_v)
            pltpu.async_copy(w_ref.at[idx_v], rows_v, sem).wait()
            pltpu.sync_copy(rows_v, out_ref.at[pl.ds(base, CHUNK)])
            return carry

        lax.fori_loop(0, NCHUNKS, chunk_body, 0)

    return body


_gather_kernel = _build()


def kernel(x, weight):
    xf = x.reshape(-1).astype(jnp.int32)
    out = _gather_kernel(xf, weight)
    return out.reshape(x.shape + (weight.shape[1],))


# speedup vs baseline: 1.5754x; 1.0169x over previous
import functools

import jax
import jax.numpy as jnp
from jax import lax
from jax.experimental import pallas as pl
from jax.experimental.pallas import tpu as pltpu
from jax.experimental.pallas import tpu_sc as plsc

EMBED_DIM = 32
B_TOTAL = 16384 * 26
NUM_CORES = 2
NUM_SUBCORES = 16
NUM_WORKERS = NUM_CORES * NUM_SUBCORES
ROWS_PER_WORKER = B_TOTAL // NUM_WORKERS
CHUNK = 416
NCHUNKS = ROWS_PER_WORKER // CHUNK
NBUF = 8
D1 = 4
D2 = NBUF - D1
NROUNDS = NCHUNKS // NBUF


def _build():
    mesh = plsc.VectorSubcoreMesh(core_axis_name="c", subcore_axis_name="s")

    scratch = (
        [pltpu.VMEM((CHUNK,), jnp.int32) for _ in range(NBUF)]
        + [pltpu.VMEM((CHUNK, EMBED_DIM), jnp.float32) for _ in range(NBUF)]
        + [pltpu.SemaphoreType.DMA for _ in range(2 * NBUF)]
    )

    @functools.partial(
        pl.kernel,
        mesh=mesh,
        out_type=jax.ShapeDtypeStruct((B_TOTAL, EMBED_DIM), jnp.float32),
        scratch_types=scratch,
        compiler_params=pltpu.CompilerParams(use_tc_tiling_on_sc=False),
    )
    def body(x_ref, w_ref, out_ref, *s):
        idx = s[0:NBUF]
        rows = s[NBUF:2 * NBUF]
        gsem = s[2 * NBUF:3 * NBUF]
        wsem = s[3 * NBUF:4 * NBUF]

        wid = lax.axis_index("s") * NUM_CORES + lax.axis_index("c")
        base0 = wid * ROWS_PER_WORKER

        def fire_gather(c, b):
            pltpu.sync_copy(x_ref.at[pl.ds(base0 + c * CHUNK, CHUNK)], idx[b])
            pltpu.async_copy(w_ref.at[idx[b]], rows[b], gsem[b])

        for j in range(D1):
            fire_gather(j, j)

        def round_body(r, carry):
            for b in range(NBUF):
                c = r * NBUF + b
                pltpu.make_async_copy(w_ref.at[idx[b]], rows[b], gsem[b]).wait()
                pltpu.async_copy(
                    rows[b], out_ref.at[pl.ds(base0 + c * CHUNK, CHUNK)], wsem[b]
                )
                b2 = (b + D1) % NBUF
                c2 = c + D1

                @pl.when(c2 >= NBUF)
                def _():
                    pltpu.make_async_copy(
                        rows[b2],
                        out_ref.at[pl.ds(base0 + (c2 - NBUF) * CHUNK, CHUNK)],
                        wsem[b2],
                    ).wait()

                @pl.when(c2 < NCHUNKS)
                def _():
                    fire_gather(c2, b2)
            return carry

        lax.fori_loop(0, NROUNDS, round_body, 0)

        for j in range(D2):
            c = NCHUNKS - D2 + j
            b = c % NBUF
            pltpu.make_async_copy(
                rows[b], out_ref.at[pl.ds(base0 + c * CHUNK, CHUNK)], wsem[b]
            ).wait()

    return body


_gather_kernel = _build()


def kernel(x, weight):
    xf = x.reshape(-1).astype(jnp.int32)
    out = _gather_kernel(xf, weight)
    return out.reshape(x.shape + (weight.shape[1],))
